# baseline (speedup 1.0000x reference)
import functools
import math

import jax
import jax.numpy as jnp
from jax import lax
from jax.experimental import pallas as pl
from jax.experimental.pallas import tpu as pltpu
from jax.experimental.pallas import tpu_sc as plsc

D_MODEL = 64
SCALE = math.sqrt(D_MODEL)

NC, NS, LANES = 2, 16, 16
NW = NC * NS

G = 128
GPC = 8
CHUNK = G * GPC


def _make_lookup(B, V):
    assert B % (NW * CHUNK) == 0
    b_per_w = B // NW
    n_chunks = b_per_w // CHUNK
    mesh = plsc.VectorSubcoreMesh(core_axis_name="c", subcore_axis_name="s")

    @functools.partial(
        pl.kernel,
        out_type=jax.ShapeDtypeStruct((B, D_MODEL), jnp.float32),
        mesh=mesh,
        scratch_types=[
            pltpu.VMEM((GPC, G), jnp.int32),
            pltpu.VMEM((CHUNK, D_MODEL), jnp.float32),
            pltpu.SemaphoreType.DMA,
        ],
        compiler_params=pltpu.CompilerParams(use_tc_tiling_on_sc=False),
    )
    def lookup(table_hbm, idx_hbm, out_hbm, idx_v, rows_v, sem):
        wid = lax.axis_index("s") * NC + lax.axis_index("c")
        row0 = wid * b_per_w

        def chunk_body(ci, _):
            base = pl.multiple_of(row0 + ci * CHUNK, CHUNK)
            jr = pl.multiple_of(base // G, GPC)
            pltpu.sync_copy(idx_hbm.at[pl.ds(jr, GPC)], idx_v)
            cps = [
                pltpu.async_copy(
                    table_hbm.at[idx_v.at[j]],
                    rows_v.at[pl.ds(j * G, G)],
                    sem,
                )
                for j in range(GPC)
            ]
            for cp in cps:
                cp.wait()

            def scale_rows(r, _):
                for u in range(2):
                    for c in range(D_MODEL // LANES):
                        sl = pl.ds(c * LANES, LANES)
                        rows_v[r * 2 + u, sl] = rows_v[r * 2 + u, sl] * SCALE
                return ()

            lax.fori_loop(0, CHUNK // 2, scale_rows, ())
            pltpu.sync_copy(rows_v, out_hbm.at[pl.ds(base, CHUNK)])
            return ()

        lax.fori_loop(0, n_chunks, chunk_body, ())

    return lookup


def kernel(x, table):
    B = x.shape[0] * x.shape[1]
    V = table.shape[0]
    idx = x.reshape(B // G, G).astype(jnp.int32)
    out = _make_lookup(B, V)(table, idx)
    return out.reshape(x.shape[0], x.shape[1], D_MODEL)


# speedup vs baseline: 1.0419x; 1.0419x over previous
import functools
import math

import jax
import jax.numpy as jnp
from jax import lax
from jax.experimental import pallas as pl
from jax.experimental.pallas import tpu as pltpu
from jax.experimental.pallas import tpu_sc as plsc

D_MODEL = 64
SCALE = math.sqrt(D_MODEL)

NC, NS, LANES = 2, 16, 16
NW = NC * NS

G = 128
GPC = 8
CHUNK = G * GPC
NQ = 4
Q = CHUNK // NQ


def _make_lookup(B, V):
    assert B % (NW * CHUNK) == 0
    b_per_w = B // NW
    n_chunks = b_per_w // CHUNK
    mesh = plsc.VectorSubcoreMesh(core_axis_name="c", subcore_axis_name="s")

    @functools.partial(
        pl.kernel,
        out_type=jax.ShapeDtypeStruct((B, D_MODEL), jnp.float32),
        mesh=mesh,
        scratch_types=[
            pltpu.VMEM((GPC, G), jnp.int32),
            pltpu.VMEM((NQ, Q, D_MODEL), jnp.float32),
        ]
        + [pltpu.SemaphoreType.DMA] * (2 * NQ),
        compiler_params=pltpu.CompilerParams(use_tc_tiling_on_sc=False),
    )
    def lookup(table_hbm, idx_hbm, out_hbm, idx_v, rows_v, *sems):
        gsem, wsem = sems[:NQ], sems[NQ:]
        wid = lax.axis_index("s") * NC + lax.axis_index("c")
        row0 = wid * b_per_w

        def chunk_body(ci, _):
            base = pl.multiple_of(row0 + ci * CHUNK, CHUNK)
            jr = pl.multiple_of(base // G, GPC)
            pltpu.sync_copy(idx_hbm.at[pl.ds(jr, GPC)], idx_v)

            cps = []
            for q in range(NQ):
                @pl.when(ci > 0)
                def _():
                    pltpu.make_async_copy(
                        rows_v.at[q], out_hbm.at[pl.ds(base, Q)], wsem[q]
                    ).wait()

                cps.append([
                    pltpu.async_copy(
                        table_hbm.at[idx_v.at[2 * q + j]],
                        rows_v.at[q].at[pl.ds(j * G, G)],
                        gsem[q],
                    )
                    for j in range(Q // G)
                ])

            for q in range(NQ):
                for cp in cps[q]:
                    cp.wait()

                def scale_rows(r, _):
                    for u in range(8):
                        for c in range(D_MODEL // LANES):
                            sl = pl.ds(c * LANES, LANES)
                            rows_v[q, r * 8 + u, sl] = (
                                rows_v[q, r * 8 + u, sl] * SCALE
                            )
                    return ()

                lax.fori_loop(0, Q // 8, scale_rows, ())
                pltpu.async_copy(
                    rows_v.at[q], out_hbm.at[pl.ds(base + q * Q, Q)], wsem[q]
                )
            return ()

        lax.fori_loop(0, n_chunks, chunk_body, ())
        for q in range(NQ):
            pltpu.make_async_copy(
                rows_v.at[q], out_hbm.at[pl.ds(0, Q)], wsem[q]
            ).wait()

    return lookup


def kernel(x, table):
    B = x.shape[0] * x.shape[1]
    V = table.shape[0]
    idx = x.reshape(B // G, G).astype(jnp.int32)
    out = _make_lookup(B, V)(table, idx)
    return out.reshape(x.shape[0], x.shape[1], D_MODEL)


# speedup vs baseline: 1.0464x; 1.0043x over previous
import functools
import math

import jax
import jax.numpy as jnp
from jax import lax
from jax.experimental import pallas as pl
from jax.experimental.pallas import tpu as pltpu
from jax.experimental.pallas import tpu_sc as plsc

D_MODEL = 64
SCALE = math.sqrt(D_MODEL)

NC, NS, LANES = 2, 16, 16
NW = NC * NS

NB = 4
RPB = 2
RPS = NB * RPB


def _make_lookup(R, S, V):
    assert R % (NW * RPS) == 0
    r_per_w = R // NW
    n_stages = r_per_w // RPS
    splits = [(0, 104), (104, 96)] if S == 200 else [(0, S)]
    mesh = plsc.VectorSubcoreMesh(core_axis_name="c", subcore_axis_name="s")

    @functools.partial(
        pl.kernel,
        out_type=jax.ShapeDtypeStruct((R, S, D_MODEL), jnp.float32),
        mesh=mesh,
        scratch_types=[
            pltpu.VMEM((RPS, S), jnp.int32),
            pltpu.VMEM((NB, RPB, S, D_MODEL), jnp.float32),
        ]
        + [pltpu.SemaphoreType.DMA] * (2 * NB),
        compiler_params=pltpu.CompilerParams(use_tc_tiling_on_sc=False),
    )
    def lookup(table_hbm, idx_hbm, out_hbm, idx_v, rows_v, *sems):
        gsem, wsem = sems[:NB], sems[NB:]
        wid = lax.axis_index("s") * NC + lax.axis_index("c")
        row0 = wid * r_per_w

        def stage_body(ci, _):
            base = pl.multiple_of(row0 + ci * RPS, RPS)
            pltpu.sync_copy(idx_hbm.at[pl.ds(base, RPS)], idx_v)

            cps = []
            for b in range(NB):
                @pl.when(ci > 0)
                def _():
                    pltpu.make_async_copy(
                        rows_v.at[b], out_hbm.at[pl.ds(base, RPB)], wsem[b]
                    ).wait()

                cps.append([
                    pltpu.async_copy(
                        table_hbm.at[idx_v.at[RPB * b + u].at[pl.ds(off, ln)]],
                        rows_v.at[b].at[u].at[pl.ds(off, ln)],
                        gsem[b],
                    )
                    for u in range(RPB)
                    for off, ln in splits
                ])

            for b in range(NB):
                for cp in cps[b]:
                    cp.wait()

                def scale_rows(q, _):
                    for u in range(RPB):
                        for i in range(4):
                            for c in range(D_MODEL // LANES):
                                sl = pl.ds(c * LANES, LANES)
                                rows_v[b, u, q * 4 + i, sl] = (
                                    rows_v[b, u, q * 4 + i, sl] * SCALE
                                )
                    return ()

                lax.fori_loop(0, S // 4, scale_rows, ())
                pltpu.async_copy(
                    rows_v.at[b],
                    out_hbm.at[pl.ds(base + b * RPB, RPB)],
                    wsem[b],
                )
            return ()

        lax.fori_loop(0, n_stages, stage_body, ())
        for b in range(NB):
            pltpu.make_async_copy(
                rows_v.at[b], out_hbm.at[pl.ds(0, RPB)], wsem[b]
            ).wait()

    return lookup


def kernel(x, table):
    R, S = x.shape
    V = table.shape[0]
    return _make_lookup(R, S, V)(table, x.astype(jnp.int32))
